# baseline (speedup 1.0000x reference)
import jax
import jax.numpy as jnp
from jax import lax
from jax.experimental import pallas as pl
from jax.experimental.pallas import tpu as pltpu
from jax.experimental.pallas import tpu_sc as plsc

N_POS = 100000
N_GENES = 2000
N_BINS = 128
NW = 32
NPAD = 102400
CHUNK = 128
CH0 = 39
CH1 = 11
SUB = (CH0 + CH1) * CHUNK
BUF = max(CH0, CH1) * CHUNK
LN2 = 0.6931471805599453


def _tables_body(uw_ref, uhm_ref, uhl_ref, fine_ref, coarse_ref):
    f32 = jnp.float32
    hi = lax.Precision.HIGHEST
    uw = uw_ref[...]
    m = jnp.max(uw, axis=1, keepdims=True)
    ew = jnp.exp(uw - m)
    w = ew / jnp.sum(ew, axis=1, keepdims=True)
    hm = jnp.exp(uhm_ref[...])
    hlast = jnp.exp(uhl_ref[...])
    r = lax.broadcasted_iota(jnp.int32, (N_BINS, N_BINS), 0)
    c = lax.broadcasted_iota(jnp.int32, (N_BINS, N_BINS), 1)
    shift = (r == c + 1).astype(f32)
    tri = (r <= c).astype(f32)
    lanes = lax.broadcasted_iota(jnp.int32, (N_GENES, N_BINS), 1)
    hshift = jnp.dot(hm, shift, precision=hi, preferred_element_type=f32)
    hr = jnp.where(lanes == N_BINS - 1, hlast, hshift)
    area = jnp.sum((hm + hr) * 0.5 * w, axis=1, keepdims=True)
    hln = hm / area
    hrn = hr / area
    inc = (hln + hrn) * 0.5 * w
    rp = lax.broadcasted_iota(jnp.int32, (N_BINS, 16 * N_BINS), 0)
    cp = lax.broadcasted_iota(jnp.int32, (N_BINS, 16 * N_BINS), 1)
    part = jnp.bitwise_and(cp, 255) // 16
    src = (cp // 256) * 16 + jnp.bitwise_and(cp, 15)
    hit = src == rp

    def onehot(k):
        return jnp.where(jnp.logical_and(hit, part == k), 1.0, 0.0).astype(f32)

    trix = tri - (r == c).astype(f32)
    pw = jnp.dot(trix, onehot(0), precision=hi, preferred_element_type=f32)
    pw = pw + onehot(1)
    ph = onehot(2) + jnp.dot(shift, onehot(3), precision=hi,
                             preferred_element_type=f32)
    pi = jnp.dot(trix, onehot(4), precision=hi, preferred_element_type=f32)
    acc = jnp.dot(w, pw, precision=hi, preferred_element_type=f32)
    acc = acc + jnp.dot(hln, ph, precision=hi, preferred_element_type=f32)
    acc = acc + jnp.dot(inc, pi, precision=hi, preferred_element_type=f32)
    cpk = lax.broadcasted_iota(jnp.int32, (N_GENES, 16 * N_BINS), 1)
    acc = jnp.where(cpk == 1855, acc + hrn[:, N_BINS - 1:N_BINS], acc)
    fine_ref[...] = acc
    rc = lax.broadcasted_iota(jnp.int32, (N_BINS, 8), 0)
    cc = lax.broadcasted_iota(jnp.int32, (N_BINS, 8), 1)
    pc = jnp.where(rc == cc * 16, 1.0, 0.0).astype(f32)
    pc = jnp.dot(trix, pc, precision=hi, preferred_element_type=f32)
    coarse_ref[...] = jnp.dot(w, pc, precision=hi, preferred_element_type=f32)


def _log16(d):
    bits = plsc.bitcast(d, jnp.int32)
    e = lax.shift_right_logical(bits, 23) - 127
    mb = jnp.bitwise_or(jnp.bitwise_and(bits, 0x7FFFFF), 0x3F800000)
    m = plsc.bitcast(mb, jnp.float32)
    big = m > 1.5
    m = jnp.where(big, m * 0.5, m)
    e = jnp.where(big, e + 1, e)
    s = (m - 1.0) / (m + 1.0)
    s2 = s * s
    p = 1.0 + s2 * (1.0 / 3.0 + s2 * (0.2 + s2 * (1.0 / 7.0)))
    return 2.0 * s * p + e.astype(jnp.float32) * LN2


def _spline_body(x_hbm, g_hbm, fine_hbm, coarse_hbm, out_hbm, logd_hbm,
                 coarse_v, x_v, g_v, o_v, l_v, idxA, idxB, rowsA, rowsB,
                 semA, semB):
    cidx = lax.axis_index("c")
    sidx = lax.axis_index("s")
    pltpu.sync_copy(coarse_hbm, coarse_v)
    lane = lax.iota(jnp.int32, 16)

    def search(off, idx_v, rows_v, sem):
        for p in range(8):
            sl = pl.ds(off + p * 16, 16)
            xs = x_v[sl]
            gb = g_v[sl] * 8
            k = jnp.zeros((16,), jnp.int32)
            for s in (4, 2, 1):
                cand = k + s
                bv = plsc.load_gather(coarse_v, [gb + cand])
                k = jnp.where(bv <= xs, cand, k)
            idx_v[pl.ds(p * 16, 16)] = gb + k
        pltpu.async_copy(fine_hbm.at[idx_v], rows_v, sem)

    def compute(off, idx_v, rows_v, sem):
        pltpu.make_async_copy(fine_hbm.at[idx_v], rows_v, sem).wait()
        for p in range(8):
            sl = pl.ds(off + p * 16, 16)
            xs = x_v[sl]
            rows = p * 16 + lane
            t = jnp.zeros((16,), jnp.int32)
            for s in (8, 4, 2, 1):
                cand = t + s
                bv = plsc.load_gather(rows_v, [rows, cand])
                t = jnp.where(bv <= xs, cand, t)
            locb = plsc.load_gather(rows_v, [rows, t])
            wv = plsc.load_gather(rows_v, [rows, t + 16])
            hl = plsc.load_gather(rows_v, [rows, t + 32])
            hr = plsc.load_gather(rows_v, [rows, t + 48])
            cf = plsc.load_gather(rows_v, [rows, t + 64])
            al = (xs - locb) / wv
            dh = hr - hl
            o_v[sl] = (0.5 * dh * wv * al + hl * wv) * al + cf
            l_v[sl] = _log16(dh * al + hl)

    def run(base, nch):
        n = nch * CHUNK
        pltpu.sync_copy(x_hbm.at[pl.ds(base, n)], x_v.at[pl.ds(0, n)])
        pltpu.sync_copy(g_hbm.at[pl.ds(base, n)], g_v.at[pl.ds(0, n)])
        search(0, idxA, rowsA, semA)

        def pair_body(i, carry):
            o0 = (2 * i) * CHUNK
            o1 = (2 * i + 1) * CHUNK
            o2 = (2 * i + 2) * CHUNK
            search(o1, idxB, rowsB, semB)
            compute(o0, idxA, rowsA, semA)
            search(o2, idxA, rowsA, semA)
            compute(o1, idxB, rowsB, semB)
            return carry

        lax.fori_loop(0, (nch - 1) // 2, pair_body, 0)
        compute((nch - 1) * CHUNK, idxA, rowsA, semA)

        pltpu.sync_copy(o_v.at[pl.ds(0, n)], out_hbm.at[pl.ds(base, n)])
        pltpu.sync_copy(l_v.at[pl.ds(0, n)], logd_hbm.at[pl.ds(base, n)])

    @pl.when(cidx == 0)
    def _():
        run(sidx * SUB, CH0)

    @pl.when(cidx == 1)
    def _():
        run(sidx * SUB + CH0 * CHUNK, CH1)


def kernel(x, local_gene_ix, unnormalized_widths, unnormalized_heights):
    f32 = jnp.float32
    uw = unnormalized_widths.astype(f32)
    uh = unnormalized_heights.astype(f32)
    uhm = uh[:, :N_BINS]
    uhl = jnp.broadcast_to(uh[:, N_BINS:], (N_GENES, N_BINS))
    fine2d, coarse2d = pl.pallas_call(
        _tables_body,
        out_shape=(jax.ShapeDtypeStruct((N_GENES, 16 * N_BINS), f32),
                   jax.ShapeDtypeStruct((N_GENES, 8), f32)),
    )(uw, uhm, uhl)
    fine = fine2d.reshape(N_GENES * 8, 256)
    coarse = coarse2d.reshape(N_GENES * 8)

    xp = jnp.concatenate([x.astype(f32), jnp.zeros((NPAD - N_POS,), f32)])
    gp = jnp.concatenate([local_gene_ix.astype(jnp.int32),
                          jnp.zeros((NPAD - N_POS,), jnp.int32)])

    mesh = plsc.VectorSubcoreMesh(core_axis_name="c", subcore_axis_name="s")
    spline = pl.kernel(
        _spline_body,
        out_type=(jax.ShapeDtypeStruct((NPAD,), f32),
                  jax.ShapeDtypeStruct((NPAD,), f32)),
        mesh=mesh,
        compiler_params=pltpu.CompilerParams(needs_layout_passes=False),
        scratch_types=[
            pltpu.VMEM((N_GENES * 8,), f32),
            pltpu.VMEM((BUF,), f32),
            pltpu.VMEM((BUF,), jnp.int32),
            pltpu.VMEM((BUF,), f32),
            pltpu.VMEM((BUF,), f32),
            pltpu.VMEM((CHUNK,), jnp.int32),
            pltpu.VMEM((CHUNK,), jnp.int32),
            pltpu.VMEM((CHUNK, 256), f32),
            pltpu.VMEM((CHUNK, 256), f32),
            pltpu.SemaphoreType.DMA,
            pltpu.SemaphoreType.DMA,
        ],
    )
    out, logd = spline(xp, gp, fine, coarse)
    return out[:N_POS], logd[:N_POS]


# speedup vs baseline: 1.2528x; 1.2528x over previous
import jax
import jax.numpy as jnp
from jax import lax
from jax.experimental import pallas as pl
from jax.experimental.pallas import tpu as pltpu
from jax.experimental.pallas import tpu_sc as plsc

N_POS = 100000
N_GENES = 2000
N_BINS = 128
NW = 32
NPAD = 102400
CHUNK = 128
CH0 = 39
CH1 = 11
SUB = (CH0 + CH1) * CHUNK
BUF = max(CH0, CH1) * CHUNK
LN2 = 0.6931471805599453


def _tables_body(uw_ref, uhm_ref, uhl_ref, fine_ref, coarse_ref):
    f32 = jnp.float32
    hi = lax.Precision.HIGHEST
    uw = uw_ref[...]
    m = jnp.max(uw, axis=1, keepdims=True)
    ew = jnp.exp(uw - m)
    w = ew / jnp.sum(ew, axis=1, keepdims=True)
    hm = jnp.exp(uhm_ref[...])
    hlast = jnp.exp(uhl_ref[...])
    r = lax.broadcasted_iota(jnp.int32, (N_BINS, N_BINS), 0)
    c = lax.broadcasted_iota(jnp.int32, (N_BINS, N_BINS), 1)
    shift = (r == c + 1).astype(f32)
    tri = (r <= c).astype(f32)
    lanes = lax.broadcasted_iota(jnp.int32, (N_GENES, N_BINS), 1)
    hshift = jnp.dot(hm, shift, precision=hi, preferred_element_type=f32)
    hr = jnp.where(lanes == N_BINS - 1, hlast, hshift)
    area = jnp.sum((hm + hr) * 0.5 * w, axis=1, keepdims=True)
    hln = hm / area
    hrn = hr / area
    inc = (hln + hrn) * 0.5 * w
    rp = lax.broadcasted_iota(jnp.int32, (N_BINS, 8 * N_BINS), 0)
    cp = lax.broadcasted_iota(jnp.int32, (N_BINS, 8 * N_BINS), 1)
    part = jnp.bitwise_and(cp, 127) // 16
    src = (cp // 128) * 16 + jnp.bitwise_and(cp, 15)
    hit = src == rp

    def onehot(k):
        return jnp.where(jnp.logical_and(hit, part == k), 1.0, 0.0).astype(f32)

    trix = tri - (r == c).astype(f32)
    pw = jnp.dot(trix, onehot(0), precision=hi, preferred_element_type=f32)
    pw = pw + onehot(1)
    ph = onehot(2) + jnp.dot(shift, onehot(3), precision=hi,
                             preferred_element_type=f32)
    pi = jnp.dot(trix, onehot(4), precision=hi, preferred_element_type=f32)
    acc = jnp.dot(w, pw, precision=hi, preferred_element_type=f32)
    acc = acc + jnp.dot(hln, ph, precision=hi, preferred_element_type=f32)
    acc = acc + jnp.dot(inc, pi, precision=hi, preferred_element_type=f32)
    cpk = lax.broadcasted_iota(jnp.int32, (N_GENES, 8 * N_BINS), 1)
    acc = jnp.where(cpk == 959, acc + hrn[:, N_BINS - 1:N_BINS], acc)
    fine_ref[...] = acc
    rc = lax.broadcasted_iota(jnp.int32, (N_BINS, 8), 0)
    cc = lax.broadcasted_iota(jnp.int32, (N_BINS, 8), 1)
    pc = jnp.where(rc == cc * 16, 1.0, 0.0).astype(f32)
    pc = jnp.dot(trix, pc, precision=hi, preferred_element_type=f32)
    coarse_ref[...] = jnp.dot(w, pc, precision=hi, preferred_element_type=f32)


def _log16(d):
    bits = plsc.bitcast(d, jnp.int32)
    e = lax.shift_right_logical(bits, 23) - 127
    mb = jnp.bitwise_or(jnp.bitwise_and(bits, 0x7FFFFF), 0x3F800000)
    m = plsc.bitcast(mb, jnp.float32)
    big = m > 1.5
    m = jnp.where(big, m * 0.5, m)
    e = jnp.where(big, e + 1, e)
    s = (m - 1.0) / (m + 1.0)
    s2 = s * s
    p = 1.0 + s2 * (1.0 / 3.0 + s2 * (0.2 + s2 * (1.0 / 7.0)))
    return 2.0 * s * p + e.astype(jnp.float32) * LN2


def _spline_body(x_hbm, g_hbm, fine_hbm, coarse_hbm, out_hbm, logd_hbm,
                 coarse_v, x_v, g_v, o_v, l_v, idxA, idxB, rowsA, rowsB,
                 semA, semB):
    cidx = lax.axis_index("c")
    sidx = lax.axis_index("s")
    pltpu.sync_copy(coarse_hbm, coarse_v)
    lane = lax.iota(jnp.int32, 16)

    def search(off, idx_v, rows_v, sem):
        for p in range(8):
            sl = pl.ds(off + p * 16, 16)
            xs = x_v[sl]
            gb = g_v[sl] * 8
            k = jnp.zeros((16,), jnp.int32)
            for s in (4, 2, 1):
                cand = k + s
                bv = plsc.load_gather(coarse_v, [gb + cand])
                k = jnp.where(bv <= xs, cand, k)
            idx_v[pl.ds(p * 16, 16)] = gb + k
        pltpu.async_copy(fine_hbm.at[idx_v], rows_v, sem)

    def compute(off, idx_v, rows_v, sem):
        pltpu.make_async_copy(fine_hbm.at[idx_v], rows_v, sem).wait()
        for p in range(8):
            sl = pl.ds(off + p * 16, 16)
            xs = x_v[sl]
            rows = p * 16 + lane
            t = jnp.zeros((16,), jnp.int32)
            for s in (8, 4, 2, 1):
                cand = t + s
                bv = plsc.load_gather(rows_v, [rows, cand])
                t = jnp.where(bv <= xs, cand, t)
            locb = plsc.load_gather(rows_v, [rows, t])
            wv = plsc.load_gather(rows_v, [rows, t + 16])
            hl = plsc.load_gather(rows_v, [rows, t + 32])
            hr = plsc.load_gather(rows_v, [rows, t + 48])
            cf = plsc.load_gather(rows_v, [rows, t + 64])
            al = (xs - locb) / wv
            dh = hr - hl
            o_v[sl] = (0.5 * dh * wv * al + hl * wv) * al + cf
            l_v[sl] = _log16(dh * al + hl)

    def run(base, nch):
        n = nch * CHUNK
        pltpu.sync_copy(x_hbm.at[pl.ds(base, n)], x_v.at[pl.ds(0, n)])
        pltpu.sync_copy(g_hbm.at[pl.ds(base, n)], g_v.at[pl.ds(0, n)])
        search(0, idxA, rowsA, semA)

        def pair_body(i, carry):
            o0 = (2 * i) * CHUNK
            o1 = (2 * i + 1) * CHUNK
            o2 = (2 * i + 2) * CHUNK
            search(o1, idxB, rowsB, semB)
            compute(o0, idxA, rowsA, semA)
            search(o2, idxA, rowsA, semA)
            compute(o1, idxB, rowsB, semB)
            return carry

        lax.fori_loop(0, (nch - 1) // 2, pair_body, 0)
        compute((nch - 1) * CHUNK, idxA, rowsA, semA)

        pltpu.sync_copy(o_v.at[pl.ds(0, n)], out_hbm.at[pl.ds(base, n)])
        pltpu.sync_copy(l_v.at[pl.ds(0, n)], logd_hbm.at[pl.ds(base, n)])

    @pl.when(cidx == 0)
    def _():
        run(sidx * SUB, CH0)

    @pl.when(cidx == 1)
    def _():
        run(sidx * SUB + CH0 * CHUNK, CH1)


def kernel(x, local_gene_ix, unnormalized_widths, unnormalized_heights):
    f32 = jnp.float32
    uw = unnormalized_widths.astype(f32)
    uh = unnormalized_heights.astype(f32)
    uhm = uh[:, :N_BINS]
    uhl = jnp.broadcast_to(uh[:, N_BINS:], (N_GENES, N_BINS))
    fine2d, coarse2d = pl.pallas_call(
        _tables_body,
        out_shape=(jax.ShapeDtypeStruct((N_GENES, 8 * N_BINS), f32),
                   jax.ShapeDtypeStruct((N_GENES, 8), f32)),
    )(uw, uhm, uhl)
    fine = fine2d.reshape(N_GENES * 8, 128)
    coarse = coarse2d.reshape(N_GENES * 8)

    xp = jnp.concatenate([x.astype(f32), jnp.zeros((NPAD - N_POS,), f32)])
    gp = jnp.concatenate([local_gene_ix.astype(jnp.int32),
                          jnp.zeros((NPAD - N_POS,), jnp.int32)])

    mesh = plsc.VectorSubcoreMesh(core_axis_name="c", subcore_axis_name="s")
    spline = pl.kernel(
        _spline_body,
        out_type=(jax.ShapeDtypeStruct((NPAD,), f32),
                  jax.ShapeDtypeStruct((NPAD,), f32)),
        mesh=mesh,
        compiler_params=pltpu.CompilerParams(needs_layout_passes=False),
        scratch_types=[
            pltpu.VMEM((N_GENES * 8,), f32),
            pltpu.VMEM((BUF,), f32),
            pltpu.VMEM((BUF,), jnp.int32),
            pltpu.VMEM((BUF,), f32),
            pltpu.VMEM((BUF,), f32),
            pltpu.VMEM((CHUNK,), jnp.int32),
            pltpu.VMEM((CHUNK,), jnp.int32),
            pltpu.VMEM((CHUNK, 128), f32),
            pltpu.VMEM((CHUNK, 128), f32),
            pltpu.SemaphoreType.DMA,
            pltpu.SemaphoreType.DMA,
        ],
    )
    out, logd = spline(xp, gp, fine, coarse)
    return out[:N_POS], logd[:N_POS]


# speedup vs baseline: 1.3288x; 1.0607x over previous
import jax
import jax.numpy as jnp
from jax import lax
from jax.experimental import pallas as pl
from jax.experimental.pallas import tpu as pltpu
from jax.experimental.pallas import tpu_sc as plsc

N_POS = 100000
N_GENES = 2000
N_BINS = 128
NW = 32
NPAD = 102400
CHUNK = 128
CH0 = 38
CH1 = 12
SUB = (CH0 + CH1) * CHUNK
BUF = max(CH0, CH1) * CHUNK
LN2 = 0.6931471805599453


def _tables_body(uw_ref, uhm_ref, uhl_ref, fine_ref, coarse_ref):
    f32 = jnp.float32
    hi = lax.Precision.HIGHEST
    uw = uw_ref[...]
    m = jnp.max(uw, axis=1, keepdims=True)
    ew = jnp.exp(uw - m)
    w = ew / jnp.sum(ew, axis=1, keepdims=True)
    hm = jnp.exp(uhm_ref[...])
    hlast = jnp.exp(uhl_ref[...])
    r = lax.broadcasted_iota(jnp.int32, (N_BINS, N_BINS), 0)
    c = lax.broadcasted_iota(jnp.int32, (N_BINS, N_BINS), 1)
    shift = (r == c + 1).astype(f32)
    tri = (r <= c).astype(f32)
    lanes = lax.broadcasted_iota(jnp.int32, (N_GENES, N_BINS), 1)
    hshift = jnp.dot(hm, shift, precision=hi, preferred_element_type=f32)
    hr = jnp.where(lanes == N_BINS - 1, hlast, hshift)
    area = jnp.sum((hm + hr) * 0.5 * w, axis=1, keepdims=True)
    hln = hm / area
    hrn = hr / area
    inc = (hln + hrn) * 0.5 * w
    rp = lax.broadcasted_iota(jnp.int32, (N_BINS, 8 * N_BINS), 0)
    cp = lax.broadcasted_iota(jnp.int32, (N_BINS, 8 * N_BINS), 1)
    part = jnp.bitwise_and(cp, 127) // 16
    src = (cp // 128) * 16 + jnp.bitwise_and(cp, 15)
    hit = src == rp

    def onehot(k):
        return jnp.where(jnp.logical_and(hit, part == k), 1.0, 0.0).astype(f32)

    trix = tri - (r == c).astype(f32)
    pw = jnp.dot(trix, onehot(0), precision=hi, preferred_element_type=f32)
    pw = pw + onehot(1)
    ph = onehot(2) + jnp.dot(shift, onehot(3), precision=hi,
                             preferred_element_type=f32)
    pi = jnp.dot(trix, onehot(4), precision=hi, preferred_element_type=f32)
    acc = jnp.dot(w, pw, precision=hi, preferred_element_type=f32)
    acc = acc + jnp.dot(hln, ph, precision=hi, preferred_element_type=f32)
    acc = acc + jnp.dot(inc, pi, precision=hi, preferred_element_type=f32)
    cpk = lax.broadcasted_iota(jnp.int32, (N_GENES, 8 * N_BINS), 1)
    acc = jnp.where(cpk == 959, acc + hrn[:, N_BINS - 1:N_BINS], acc)
    for cb in range(8):
        fine_ref[cb] = acc[:, N_BINS * cb:N_BINS * (cb + 1)]
    rc = lax.broadcasted_iota(jnp.int32, (N_BINS, 8), 0)
    cc = lax.broadcasted_iota(jnp.int32, (N_BINS, 8), 1)
    pc = jnp.where(rc == cc * 16, 1.0, 0.0).astype(f32)
    pc = jnp.dot(trix, pc, precision=hi, preferred_element_type=f32)
    coarse_ref[...] = jnp.dot(w, pc, precision=hi, preferred_element_type=f32)


def _log16(d):
    bits = plsc.bitcast(d, jnp.int32)
    e = lax.shift_right_logical(bits, 23) - 127
    mb = jnp.bitwise_or(jnp.bitwise_and(bits, 0x7FFFFF), 0x3F800000)
    m = plsc.bitcast(mb, jnp.float32)
    big = m > 1.5
    m = jnp.where(big, m * 0.5, m)
    e = jnp.where(big, e + 1, e)
    s = (m - 1.0) / (m + 1.0)
    s2 = s * s
    p = 1.0 + s2 * (1.0 / 3.0 + s2 * (0.2 + s2 * (1.0 / 7.0)))
    return 2.0 * s * p + e.astype(jnp.float32) * LN2


def _spline_body(x_hbm, g_hbm, fine_hbm, coarse_hbm, out_hbm, logd_hbm,
                 coarse_v, x_v, g_v, o_v, l_v, idxA, idxB, rowsA, rowsB,
                 semA, semB):
    cidx = lax.axis_index("c")
    sidx = lax.axis_index("s")
    pltpu.sync_copy(coarse_hbm, coarse_v)
    lane = lax.iota(jnp.int32, 16)

    def search(off, idx_v, rows_v, sem):
        for p in range(8):
            sl = pl.ds(off + p * 16, 16)
            xs = x_v[sl]
            gs = g_v[sl]
            gb = gs * 8
            k = jnp.zeros((16,), jnp.int32)
            for s in (4, 2, 1):
                cand = k + s
                bv = plsc.load_gather(coarse_v, [gb + cand])
                k = jnp.where(bv <= xs, cand, k)
            idx_v[pl.ds(p * 16, 16)] = k * N_GENES + gs
        pltpu.async_copy(fine_hbm.at[idx_v], rows_v, sem)

    def compute(off, idx_v, rows_v, sem):
        pltpu.make_async_copy(fine_hbm.at[idx_v], rows_v, sem).wait()
        for p in range(8):
            sl = pl.ds(off + p * 16, 16)
            xs = x_v[sl]
            rows = p * 16 + lane
            t = jnp.zeros((16,), jnp.int32)
            for s in (8, 4, 2, 1):
                cand = t + s
                bv = plsc.load_gather(rows_v, [rows, cand])
                t = jnp.where(bv <= xs, cand, t)
            locb = plsc.load_gather(rows_v, [rows, t])
            wv = plsc.load_gather(rows_v, [rows, t + 16])
            hl = plsc.load_gather(rows_v, [rows, t + 32])
            hr = plsc.load_gather(rows_v, [rows, t + 48])
            cf = plsc.load_gather(rows_v, [rows, t + 64])
            al = (xs - locb) / wv
            dh = hr - hl
            o_v[sl] = (0.5 * dh * wv * al + hl * wv) * al + cf
            l_v[sl] = _log16(dh * al + hl)

    def run(base, nch):
        n = nch * CHUNK
        pltpu.sync_copy(x_hbm.at[pl.ds(base, n)], x_v.at[pl.ds(0, n)])
        pltpu.sync_copy(g_hbm.at[pl.ds(base, n)], g_v.at[pl.ds(0, n)])
        search(0, idxA, rowsA, semA)

        def pair_body(i, carry):
            o0 = (2 * i) * CHUNK
            o1 = (2 * i + 1) * CHUNK
            o2 = (2 * i + 2) * CHUNK
            search(o1, idxB, rowsB, semB)
            compute(o0, idxA, rowsA, semA)
            search(o2, idxA, rowsA, semA)
            compute(o1, idxB, rowsB, semB)
            return carry

        lax.fori_loop(0, (nch - 1) // 2, pair_body, 0)
        if nch % 2 == 1:
            compute((nch - 1) * CHUNK, idxA, rowsA, semA)
        else:
            search((nch - 1) * CHUNK, idxB, rowsB, semB)
            compute((nch - 2) * CHUNK, idxA, rowsA, semA)
            compute((nch - 1) * CHUNK, idxB, rowsB, semB)

        pltpu.sync_copy(o_v.at[pl.ds(0, n)], out_hbm.at[pl.ds(base, n)])
        pltpu.sync_copy(l_v.at[pl.ds(0, n)], logd_hbm.at[pl.ds(base, n)])

    @pl.when(cidx == 0)
    def _():
        run(sidx * SUB, CH0)

    @pl.when(cidx == 1)
    def _():
        run(sidx * SUB + CH0 * CHUNK, CH1)


def kernel(x, local_gene_ix, unnormalized_widths, unnormalized_heights):
    f32 = jnp.float32
    uw = unnormalized_widths.astype(f32)
    uh = unnormalized_heights.astype(f32)
    uhm = uh[:, :N_BINS]
    uhl = jnp.broadcast_to(uh[:, N_BINS:], (N_GENES, N_BINS))
    fine3d, coarse2d = pl.pallas_call(
        _tables_body,
        out_shape=(jax.ShapeDtypeStruct((8, N_GENES, N_BINS), f32),
                   jax.ShapeDtypeStruct((N_GENES, 8), f32)),
    )(uw, uhm, uhl)
    fine = fine3d.reshape(8 * N_GENES, 128)
    coarse = coarse2d.reshape(N_GENES * 8)

    xp = jnp.concatenate([x.astype(f32), jnp.zeros((NPAD - N_POS,), f32)])
    gp = jnp.concatenate([local_gene_ix.astype(jnp.int32),
                          jnp.zeros((NPAD - N_POS,), jnp.int32)])

    mesh = plsc.VectorSubcoreMesh(core_axis_name="c", subcore_axis_name="s")
    spline = pl.kernel(
        _spline_body,
        out_type=(jax.ShapeDtypeStruct((NPAD,), f32),
                  jax.ShapeDtypeStruct((NPAD,), f32)),
        mesh=mesh,
        compiler_params=pltpu.CompilerParams(needs_layout_passes=False),
        scratch_types=[
            pltpu.VMEM((N_GENES * 8,), f32),
            pltpu.VMEM((BUF,), f32),
            pltpu.VMEM((BUF,), jnp.int32),
            pltpu.VMEM((BUF,), f32),
            pltpu.VMEM((BUF,), f32),
            pltpu.VMEM((CHUNK,), jnp.int32),
            pltpu.VMEM((CHUNK,), jnp.int32),
            pltpu.VMEM((CHUNK, 128), f32),
            pltpu.VMEM((CHUNK, 128), f32),
            pltpu.SemaphoreType.DMA,
            pltpu.SemaphoreType.DMA,
        ],
    )
    out, logd = spline(xp, gp, fine, coarse)
    return out[:N_POS], logd[:N_POS]


# speedup vs baseline: 1.4025x; 1.0554x over previous
import jax
import jax.numpy as jnp
from jax import lax
from jax.experimental import pallas as pl
from jax.experimental.pallas import tpu as pltpu
from jax.experimental.pallas import tpu_sc as plsc

N_POS = 100000
N_GENES = 2000
N_BINS = 128
NW = 32
NPAD = 102400
CHUNK = 128
CH0 = 38
CH1 = 12
SUB = (CH0 + CH1) * CHUNK
BUF = max(CH0, CH1) * CHUNK
LN2 = 0.6931471805599453


def _tables_body(uw_ref, uhm_ref, uhl_ref, fine_ref, coarse_ref):
    f32 = jnp.float32
    hi = lax.Precision.HIGHEST
    uw = uw_ref[...]
    m = jnp.max(uw, axis=1, keepdims=True)
    ew = jnp.exp(uw - m)
    w = ew / jnp.sum(ew, axis=1, keepdims=True)
    hm = jnp.exp(uhm_ref[...])
    hlast = jnp.exp(uhl_ref[...])
    r = lax.broadcasted_iota(jnp.int32, (N_BINS, N_BINS), 0)
    c = lax.broadcasted_iota(jnp.int32, (N_BINS, N_BINS), 1)
    shift = (r == c + 1).astype(f32)
    tri = (r <= c).astype(f32)
    lanes = lax.broadcasted_iota(jnp.int32, (N_GENES, N_BINS), 1)
    hshift = jnp.dot(hm, shift, precision=hi, preferred_element_type=f32)
    hr = jnp.where(lanes == N_BINS - 1, hlast, hshift)
    area = jnp.sum((hm + hr) * 0.5 * w, axis=1, keepdims=True)
    hln = hm / area
    hrn = hr / area
    inc = (hln + hrn) * 0.5 * w
    rp = lax.broadcasted_iota(jnp.int32, (N_BINS, 8 * N_BINS), 0)
    cp = lax.broadcasted_iota(jnp.int32, (N_BINS, 8 * N_BINS), 1)
    part = jnp.bitwise_and(cp, 127) // 16
    src = (cp // 128) * 16 + jnp.bitwise_and(cp, 15)
    hit = src == rp

    def onehot(k):
        return jnp.where(jnp.logical_and(hit, part == k), 1.0, 0.0).astype(f32)

    trix = tri - (r == c).astype(f32)
    pw = jnp.dot(trix, onehot(0), precision=hi, preferred_element_type=f32)
    pw = pw + onehot(1)
    ph = onehot(2) + jnp.dot(shift, onehot(3), precision=hi,
                             preferred_element_type=f32)
    pi = jnp.dot(trix, onehot(4), precision=hi, preferred_element_type=f32)

    bf16 = jnp.bfloat16

    def split3(v):
        v0 = v.astype(bf16)
        r1 = v - v0.astype(f32)
        v1 = r1.astype(bf16)
        v2 = (r1 - v1.astype(f32)).astype(bf16)
        return v0, v1, v2

    def dot3(v, pconst):
        pb = pconst.astype(bf16)
        out = jnp.zeros((N_GENES, 8 * N_BINS), f32)
        for part in split3(v):
            out = out + jnp.dot(part, pb, preferred_element_type=f32)
        return out

    acc = dot3(w, pw) + dot3(hln, ph) + dot3(inc, pi)
    cpk = lax.broadcasted_iota(jnp.int32, (N_GENES, 8 * N_BINS), 1)
    acc = jnp.where(cpk == 959, acc + hrn[:, N_BINS - 1:N_BINS], acc)
    for cb in range(8):
        fine_ref[cb] = acc[:, N_BINS * cb:N_BINS * (cb + 1)]
    rc = lax.broadcasted_iota(jnp.int32, (N_BINS, 8), 0)
    cc = lax.broadcasted_iota(jnp.int32, (N_BINS, 8), 1)
    pc = jnp.where(rc == cc * 16, 1.0, 0.0).astype(f32)
    pc = jnp.dot(trix, pc, precision=hi, preferred_element_type=f32)
    coarse_ref[...] = jnp.dot(w, pc, precision=hi, preferred_element_type=f32)


def _log16(d):
    bits = plsc.bitcast(d, jnp.int32)
    e = lax.shift_right_logical(bits, 23) - 127
    mb = jnp.bitwise_or(jnp.bitwise_and(bits, 0x7FFFFF), 0x3F800000)
    m = plsc.bitcast(mb, jnp.float32)
    big = m > 1.5
    m = jnp.where(big, m * 0.5, m)
    e = jnp.where(big, e + 1, e)
    s = (m - 1.0) / (m + 1.0)
    s2 = s * s
    p = 1.0 + s2 * (1.0 / 3.0 + s2 * (0.2 + s2 * (1.0 / 7.0)))
    return 2.0 * s * p + e.astype(jnp.float32) * LN2


def _spline_body(x_hbm, g_hbm, fine_hbm, coarse_hbm, out_hbm, logd_hbm,
                 coarse_v, x_v, g_v, o_v, l_v, idxA, idxB, rowsA, rowsB,
                 semA, semB):
    cidx = lax.axis_index("c")
    sidx = lax.axis_index("s")
    pltpu.sync_copy(coarse_hbm, coarse_v)
    lane = lax.iota(jnp.int32, 16)

    def search(off, idx_v, rows_v, sem):
        for p in range(8):
            sl = pl.ds(off + p * 16, 16)
            xs = x_v[sl]
            gs = g_v[sl]
            gb = gs * 8
            k = jnp.zeros((16,), jnp.int32)
            for s in (4, 2, 1):
                cand = k + s
                bv = plsc.load_gather(coarse_v, [gb + cand])
                k = jnp.where(bv <= xs, cand, k)
            idx_v[pl.ds(p * 16, 16)] = k * N_GENES + gs
        pltpu.async_copy(fine_hbm.at[idx_v], rows_v, sem)

    def compute(off, idx_v, rows_v, sem):
        pltpu.make_async_copy(fine_hbm.at[idx_v], rows_v, sem).wait()
        for p in range(8):
            sl = pl.ds(off + p * 16, 16)
            xs = x_v[sl]
            rows = p * 16 + lane
            t = jnp.zeros((16,), jnp.int32)
            for s in (8, 4, 2, 1):
                cand = t + s
                bv = plsc.load_gather(rows_v, [rows, cand])
                t = jnp.where(bv <= xs, cand, t)
            locb = plsc.load_gather(rows_v, [rows, t])
            wv = plsc.load_gather(rows_v, [rows, t + 16])
            hl = plsc.load_gather(rows_v, [rows, t + 32])
            hr = plsc.load_gather(rows_v, [rows, t + 48])
            cf = plsc.load_gather(rows_v, [rows, t + 64])
            al = (xs - locb) / wv
            dh = hr - hl
            o_v[sl] = (0.5 * dh * wv * al + hl * wv) * al + cf
            l_v[sl] = _log16(dh * al + hl)

    def run(base, nch):
        n = nch * CHUNK
        pltpu.sync_copy(x_hbm.at[pl.ds(base, n)], x_v.at[pl.ds(0, n)])
        pltpu.sync_copy(g_hbm.at[pl.ds(base, n)], g_v.at[pl.ds(0, n)])
        search(0, idxA, rowsA, semA)

        def pair_body(i, carry):
            o0 = (2 * i) * CHUNK
            o1 = (2 * i + 1) * CHUNK
            o2 = (2 * i + 2) * CHUNK
            search(o1, idxB, rowsB, semB)
            compute(o0, idxA, rowsA, semA)
            search(o2, idxA, rowsA, semA)
            compute(o1, idxB, rowsB, semB)
            return carry

        lax.fori_loop(0, (nch - 1) // 2, pair_body, 0)
        if nch % 2 == 1:
            compute((nch - 1) * CHUNK, idxA, rowsA, semA)
        else:
            search((nch - 1) * CHUNK, idxB, rowsB, semB)
            compute((nch - 2) * CHUNK, idxA, rowsA, semA)
            compute((nch - 1) * CHUNK, idxB, rowsB, semB)

        pltpu.sync_copy(o_v.at[pl.ds(0, n)], out_hbm.at[pl.ds(base, n)])
        pltpu.sync_copy(l_v.at[pl.ds(0, n)], logd_hbm.at[pl.ds(base, n)])

    @pl.when(cidx == 0)
    def _():
        run(sidx * SUB, CH0)

    @pl.when(cidx == 1)
    def _():
        run(sidx * SUB + CH0 * CHUNK, CH1)


def kernel(x, local_gene_ix, unnormalized_widths, unnormalized_heights):
    f32 = jnp.float32
    uw = unnormalized_widths.astype(f32)
    uh = unnormalized_heights.astype(f32)
    uhm = uh[:, :N_BINS]
    uhl = jnp.broadcast_to(uh[:, N_BINS:], (N_GENES, N_BINS))
    fine3d, coarse2d = pl.pallas_call(
        _tables_body,
        out_shape=(jax.ShapeDtypeStruct((8, N_GENES, N_BINS), f32),
                   jax.ShapeDtypeStruct((N_GENES, 8), f32)),
    )(uw, uhm, uhl)
    fine = fine3d.reshape(8 * N_GENES, 128)
    coarse = coarse2d.reshape(N_GENES * 8)

    xp = jnp.concatenate([x.astype(f32), jnp.zeros((NPAD - N_POS,), f32)])
    gp = jnp.concatenate([local_gene_ix.astype(jnp.int32),
                          jnp.zeros((NPAD - N_POS,), jnp.int32)])

    mesh = plsc.VectorSubcoreMesh(core_axis_name="c", subcore_axis_name="s")
    spline = pl.kernel(
        _spline_body,
        out_type=(jax.ShapeDtypeStruct((NPAD,), f32),
                  jax.ShapeDtypeStruct((NPAD,), f32)),
        mesh=mesh,
        compiler_params=pltpu.CompilerParams(needs_layout_passes=False),
        scratch_types=[
            pltpu.VMEM((N_GENES * 8,), f32),
            pltpu.VMEM((BUF,), f32),
            pltpu.VMEM((BUF,), jnp.int32),
            pltpu.VMEM((BUF,), f32),
            pltpu.VMEM((BUF,), f32),
            pltpu.VMEM((CHUNK,), jnp.int32),
            pltpu.VMEM((CHUNK,), jnp.int32),
            pltpu.VMEM((CHUNK, 128), f32),
            pltpu.VMEM((CHUNK, 128), f32),
            pltpu.SemaphoreType.DMA,
            pltpu.SemaphoreType.DMA,
        ],
    )
    out, logd = spline(xp, gp, fine, coarse)
    return out[:N_POS], logd[:N_POS]
